# baseline (speedup 1.0000x reference)
import jax
import jax.numpy as jnp
from jax import lax
from jax.experimental import pallas as pl
from jax.experimental.pallas import tpu as pltpu
from jax.experimental.pallas import tpu_sc as plsc

_N = 100000
_D = 128
_L = 16
_NC = 2
_NS = 16
_CHUNK = 128
_STRIPE_CHUNKS = 50
_STRIPE = _STRIPE_CHUNKS * _CHUNK
_N_PAD = _NS * _STRIPE
_TAIL = _N % _CHUNK
_A = 25
_B = _STRIPE_CHUNKS - _A


_TROWS = 512 // _NS


def _sc_body(in_deg, out_deg, in_tab, out_tab, out,
             idx_a, idx_b, sh_a, sh_b,
             ra0, ra1, ra2, rb0, rb1, rb2,
             sga0, sga1, sga2, sgb0, sgb1, sgb2, sw0, sw1, sw2, sst):
    cid = lax.axis_index("c")
    sid = lax.axis_index("s")
    ras = (ra0, ra1, ra2)
    rbs = (rb0, rb1, rb2)
    sgas = (sga0, sga1, sga2)
    sgbs = (sgb0, sgb1, sgb2)
    sws = (sw0, sw1, sw2)

    base = sid * _STRIPE + cid * (_A * _CHUNK)
    nrows = _A * _CHUNK
    tsl = pl.ds(sid * _TROWS, _TROWS)
    isl = pl.ds(base, nrows)
    osl = pl.ds(0, nrows)
    pltpu.async_copy(in_tab.at[tsl], sh_a.at[tsl], sst)
    pltpu.async_copy(out_tab.at[tsl], sh_b.at[tsl], sst)
    pltpu.async_copy(in_deg.at[isl], idx_a.at[osl], sst)
    pltpu.async_copy(out_deg.at[isl], idx_b.at[osl], sst)
    pltpu.make_async_copy(in_tab.at[tsl], sh_a.at[tsl], sst).wait()
    pltpu.make_async_copy(out_tab.at[tsl], sh_b.at[tsl], sst).wait()
    pltpu.make_async_copy(in_deg.at[isl], idx_a.at[osl], sst).wait()
    pltpu.make_async_copy(out_deg.at[isl], idx_b.at[osl], sst).wait()
    plsc.subcore_barrier()

    def issue_gather(c, s):
        sl = pl.ds(c * _CHUNK, _CHUNK)
        pltpu.async_copy(sh_a.at[idx_a.at[sl]], ras[s], sgas[s])
        pltpu.async_copy(sh_b.at[idx_b.at[sl]], rbs[s], sgbs[s])

    def wait_gather(s):
        sl = pl.ds(0, _CHUNK)
        pltpu.make_async_copy(sh_a.at[idx_a.at[sl]], ras[s], sgas[s]).wait()
        pltpu.make_async_copy(sh_b.at[idx_b.at[sl]], rbs[s], sgbs[s]).wait()

    def issue_write(base, c, s):
        gbase = base + c * _CHUNK

        @pl.when(gbase + _CHUNK <= _N)
        def _():
            pltpu.async_copy(ras[s], out.at[pl.ds(gbase, _CHUNK)], sws[s])

        @pl.when(jnp.logical_and(gbase < _N, gbase + _CHUNK > _N))
        def _():
            pltpu.async_copy(ras[s].at[pl.ds(0, _TAIL)],
                             out.at[pl.ds(gbase, _TAIL)], sws[s])

    def wait_write(base, c, s, extra_pred):
        gbase = base + c * _CHUNK
        p_full = jnp.logical_and(extra_pred, gbase + _CHUNK <= _N)
        p_part = jnp.logical_and(
            extra_pred, jnp.logical_and(gbase < _N, gbase + _CHUNK > _N))

        @pl.when(p_full)
        def _():
            pltpu.make_async_copy(ras[s], out.at[pl.ds(gbase, _CHUNK)],
                                  sws[s]).wait()

        @pl.when(p_part)
        def _():
            pltpu.make_async_copy(ras[s].at[pl.ds(0, _TAIL)],
                                  out.at[pl.ds(gbase, _TAIL)], sws[s]).wait()

    def accumulate(s):
        ra, rb = ras[s], rbs[s]

        @plsc.parallel_loop(0, _CHUNK, unroll=8)
        def _(r):
            for j in range(_D // _L):
                sl = pl.ds(j * _L, _L)
                plsc.addupdate(ra.at[r, sl], rb[r, sl])

    true_pred = jnp.bool_(True)

    def pipeline(nc, base):
        n_triples = (nc - 1) // 3
        issue_gather(0, 0)

        def body(j, carry):
            for s in range(3):
                c = 3 * j + s
                sn = (s + 1) % 3
                wait_write(base, c - 2, sn, c >= 2)
                issue_gather(c + 1, sn)
                wait_gather(s)
                accumulate(s)
                issue_write(base, c, s)
            return carry

        lax.fori_loop(0, n_triples, body, 0)

        for c in range(3 * n_triples, nc):
            s = c % 3
            sn = (s + 1) % 3
            if c + 1 < nc:
                wait_write(base, c - 2, sn, true_pred)
                issue_gather(c + 1, sn)
            wait_gather(s)
            accumulate(s)
            issue_write(base, c, s)

        for c in range(max(0, nc - 3), nc):
            wait_write(base, c, c % 3, true_pred)

    pipeline(_A, base)


def kernel(in_degrees, out_degrees, in_table, out_table):
    pad = _N_PAD - _N
    zeros = jnp.zeros((pad,), jnp.int32)
    in_idx = jnp.concatenate([in_degrees.astype(jnp.int32), zeros])
    out_idx = jnp.concatenate([out_degrees.astype(jnp.int32), zeros])

    mesh = plsc.VectorSubcoreMesh(core_axis_name="c", subcore_axis_name="s")
    max_rows = max(_A, _B) * _CHUNK
    return pl.kernel(
        _sc_body,
        out_type=jax.ShapeDtypeStruct((_N, _D), jnp.float32),
        mesh=mesh,
        scratch_types=[
            pltpu.VMEM((max_rows,), jnp.int32),
            pltpu.VMEM((max_rows,), jnp.int32),
            pltpu.VMEM_SHARED((512, _D), jnp.float32),
            pltpu.VMEM_SHARED((512, _D), jnp.float32),
            pltpu.VMEM((_CHUNK, _D), jnp.float32),
            pltpu.VMEM((_CHUNK, _D), jnp.float32),
            pltpu.VMEM((_CHUNK, _D), jnp.float32),
            pltpu.VMEM((_CHUNK, _D), jnp.float32),
            pltpu.VMEM((_CHUNK, _D), jnp.float32),
            pltpu.VMEM((_CHUNK, _D), jnp.float32),
            pltpu.SemaphoreType.DMA,
            pltpu.SemaphoreType.DMA,
            pltpu.SemaphoreType.DMA,
            pltpu.SemaphoreType.DMA,
            pltpu.SemaphoreType.DMA,
            pltpu.SemaphoreType.DMA,
            pltpu.SemaphoreType.DMA,
            pltpu.SemaphoreType.DMA,
            pltpu.SemaphoreType.DMA,
            pltpu.SemaphoreType.DMA,
        ],
    )(in_idx, out_idx, in_table, out_table)


# speedup vs baseline: 1.0181x; 1.0181x over previous
import jax
import jax.numpy as jnp
from jax import lax
from jax.experimental import pallas as pl
from jax.experimental.pallas import tpu as pltpu
from jax.experimental.pallas import tpu_sc as plsc

_N = 100000
_D = 128
_L = 16
_NC = 2
_NS = 16
_CHUNK = 128
_STRIPE_CHUNKS = 50
_STRIPE = _STRIPE_CHUNKS * _CHUNK
_N_PAD = _NS * _STRIPE
_TAIL = _N % _CHUNK
_A = 25
_B = _STRIPE_CHUNKS - _A


_TROWS = 512 // _NS


def _sc_body(in_deg, out_deg, in_tab, out_tab, out,
             idx_a, idx_b, sh_a, sh_b,
             ra0, ra1, ra2, rb0, rb1, rb2,
             sga0, sga1, sga2, sgb0, sgb1, sgb2, sw0, sw1, sw2, sst):
    cid = lax.axis_index("c")
    sid = lax.axis_index("s")
    ras = (ra0, ra1, ra2)
    rbs = (rb0, rb1, rb2)
    sgas = (sga0, sga1, sga2)
    sgbs = (sgb0, sgb1, sgb2)
    sws = (sw0, sw1, sw2)

    base = sid * _STRIPE + cid * (_A * _CHUNK)
    nrows = _A * _CHUNK
    tsl = pl.ds(sid * _TROWS, _TROWS)
    isl = pl.ds(base, nrows)
    osl = pl.ds(0, nrows)
    pltpu.async_copy(in_tab.at[tsl], sh_a.at[tsl], sst)
    pltpu.async_copy(out_tab.at[tsl], sh_b.at[tsl], sst)
    pltpu.async_copy(in_deg.at[isl], idx_a.at[osl], sst)
    pltpu.async_copy(out_deg.at[isl], idx_b.at[osl], sst)
    pltpu.make_async_copy(in_tab.at[tsl], sh_a.at[tsl], sst).wait()
    pltpu.make_async_copy(out_tab.at[tsl], sh_b.at[tsl], sst).wait()
    pltpu.make_async_copy(in_deg.at[isl], idx_a.at[osl], sst).wait()
    pltpu.make_async_copy(out_deg.at[isl], idx_b.at[osl], sst).wait()
    plsc.subcore_barrier()

    def issue_gather(c, s):
        sl = pl.ds(c * _CHUNK, _CHUNK)
        pltpu.async_copy(sh_a.at[idx_a.at[sl]], ras[s], sgas[s])
        pltpu.async_copy(sh_b.at[idx_b.at[sl]], rbs[s], sgbs[s])

    def wait_gather(s):
        sl = pl.ds(0, _CHUNK)
        pltpu.make_async_copy(sh_a.at[idx_a.at[sl]], ras[s], sgas[s]).wait()
        pltpu.make_async_copy(sh_b.at[idx_b.at[sl]], rbs[s], sgbs[s]).wait()

    def issue_write(base, c, s):
        gbase = base + c * _CHUNK

        @pl.when(gbase + _CHUNK <= _N)
        def _():
            pltpu.async_copy(ras[s], out.at[pl.ds(gbase, _CHUNK)], sws[s])

        @pl.when(jnp.logical_and(gbase < _N, gbase + _CHUNK > _N))
        def _():
            pltpu.async_copy(ras[s].at[pl.ds(0, _TAIL)],
                             out.at[pl.ds(gbase, _TAIL)], sws[s])

    def wait_write(base, c, s, extra_pred):
        gbase = base + c * _CHUNK
        p_full = jnp.logical_and(extra_pred, gbase + _CHUNK <= _N)
        p_part = jnp.logical_and(
            extra_pred, jnp.logical_and(gbase < _N, gbase + _CHUNK > _N))

        @pl.when(p_full)
        def _():
            pltpu.make_async_copy(ras[s], out.at[pl.ds(gbase, _CHUNK)],
                                  sws[s]).wait()

        @pl.when(p_part)
        def _():
            pltpu.make_async_copy(ras[s].at[pl.ds(0, _TAIL)],
                                  out.at[pl.ds(gbase, _TAIL)], sws[s]).wait()

    def accumulate(s):
        ra, rb = ras[s], rbs[s]

        @plsc.parallel_loop(0, _CHUNK, unroll=4)
        def _(r):
            for j in range(_D // _L):
                sl = pl.ds(j * _L, _L)
                plsc.addupdate(ra.at[r, sl], rb[r, sl])

    true_pred = jnp.bool_(True)

    def pipeline(nc, base):
        n_triples = (nc - 1) // 3
        issue_gather(0, 0)

        def body(j, carry):
            for s in range(3):
                c = 3 * j + s
                sn = (s + 1) % 3
                wait_write(base, c - 2, sn, c >= 2)
                issue_gather(c + 1, sn)
                wait_gather(s)
                accumulate(s)
                issue_write(base, c, s)
            return carry

        lax.fori_loop(0, n_triples, body, 0)

        for c in range(3 * n_triples, nc):
            s = c % 3
            sn = (s + 1) % 3
            if c + 1 < nc:
                wait_write(base, c - 2, sn, true_pred)
                issue_gather(c + 1, sn)
            wait_gather(s)
            accumulate(s)
            issue_write(base, c, s)

        for c in range(max(0, nc - 3), nc):
            wait_write(base, c, c % 3, true_pred)

    pipeline(_A, base)


def kernel(in_degrees, out_degrees, in_table, out_table):
    pad = _N_PAD - _N
    zeros = jnp.zeros((pad,), jnp.int32)
    in_idx = jnp.concatenate([in_degrees.astype(jnp.int32), zeros])
    out_idx = jnp.concatenate([out_degrees.astype(jnp.int32), zeros])

    mesh = plsc.VectorSubcoreMesh(core_axis_name="c", subcore_axis_name="s")
    max_rows = max(_A, _B) * _CHUNK
    return pl.kernel(
        _sc_body,
        out_type=jax.ShapeDtypeStruct((_N, _D), jnp.float32),
        mesh=mesh,
        scratch_types=[
            pltpu.VMEM((max_rows,), jnp.int32),
            pltpu.VMEM((max_rows,), jnp.int32),
            pltpu.VMEM_SHARED((512, _D), jnp.float32),
            pltpu.VMEM_SHARED((512, _D), jnp.float32),
            pltpu.VMEM((_CHUNK, _D), jnp.float32),
            pltpu.VMEM((_CHUNK, _D), jnp.float32),
            pltpu.VMEM((_CHUNK, _D), jnp.float32),
            pltpu.VMEM((_CHUNK, _D), jnp.float32),
            pltpu.VMEM((_CHUNK, _D), jnp.float32),
            pltpu.VMEM((_CHUNK, _D), jnp.float32),
            pltpu.SemaphoreType.DMA,
            pltpu.SemaphoreType.DMA,
            pltpu.SemaphoreType.DMA,
            pltpu.SemaphoreType.DMA,
            pltpu.SemaphoreType.DMA,
            pltpu.SemaphoreType.DMA,
            pltpu.SemaphoreType.DMA,
            pltpu.SemaphoreType.DMA,
            pltpu.SemaphoreType.DMA,
            pltpu.SemaphoreType.DMA,
        ],
    )(in_idx, out_idx, in_table, out_table)
